# baseline (speedup 1.0000x reference)
import math

import jax
import jax.numpy as jnp
from jax import lax
from jax.experimental import pallas as pl
from jax.experimental.pallas import tpu as pltpu
from jax.experimental.pallas import tpu_sc as plsc

VOCAB = 1000000
D = 64
ROWS = 16384
COLS = 200
B = ROWS * COLS

NC = 2
NS = 16
NW = NC * NS
PER_W = B // NW
C = 512
NCH = PER_W // C
SCALE = math.sqrt(D)

_mesh = plsc.VectorSubcoreMesh(
    core_axis_name="c", subcore_axis_name="s", num_cores=NC, num_subcores=NS
)


def _body(table_hbm, idx_hbm, out_hbm, idx_v, rows_v, gsem):
    wid = lax.axis_index("s") * NC + lax.axis_index("c")
    base = wid * PER_W

    @pl.loop(0, NCH)
    def _chunk(g):
        off = base + g * C
        pltpu.sync_copy(idx_hbm.at[pl.ds(off, C)], idx_v)
        pltpu.async_copy(table_hbm.at[idx_v], rows_v, gsem).wait()

        @pl.loop(0, C)
        def _scale(r):
            for k in range(D // 16):
                sl = pl.ds(k * 16, 16)
                rows_v[r, sl] = rows_v[r, sl] * SCALE

        pltpu.sync_copy(rows_v, out_hbm.at[pl.ds(off, C)])


_lookup = pl.kernel(
    _body,
    out_type=jax.ShapeDtypeStruct((B, D), jnp.float32),
    mesh=_mesh,
    scratch_types=[
        pltpu.VMEM((C,), jnp.int32),
        pltpu.VMEM((C, D), jnp.float32),
        pltpu.SemaphoreType.DMA,
    ],
    compiler_params=pltpu.CompilerParams(use_tc_tiling_on_sc=False),
)


@jax.jit
def kernel(x, table):
    flat = x.reshape(B)
    out = _lookup(table, flat)
    return out.reshape(ROWS, COLS, D)


# speedup vs baseline: 1.1958x; 1.1958x over previous
import math

import jax
import jax.numpy as jnp
from jax import lax
from jax.experimental import pallas as pl
from jax.experimental.pallas import tpu as pltpu
from jax.experimental.pallas import tpu_sc as plsc

VOCAB = 1000000
D = 64
ROWS = 16384
COLS = 200
B = ROWS * COLS

NC = 2
NS = 16
NW = NC * NS
PER_W = B // NW
C = 512
NCH = PER_W // C
SCALE = math.sqrt(D)

_mesh = plsc.VectorSubcoreMesh(
    core_axis_name="c", subcore_axis_name="s", num_cores=NC, num_subcores=NS
)


def _body(table_hbm, idx_hbm, out_hbm, idx0, idx1, rows0, rows1,
          gsem0, gsem1, ssem0, ssem1):
    wid = lax.axis_index("s") * NC + lax.axis_index("c")
    base = wid * PER_W
    idx = [idx0, idx1]
    rows = [rows0, rows1]
    gsem = [gsem0, gsem1]
    ssem = [ssem0, ssem1]

    def start_gather(ch, b):
        off = base + ch * C
        pltpu.sync_copy(idx_hbm.at[pl.ds(off, C)], idx[b])
        pltpu.async_copy(table_hbm.at[idx[b]], rows[b], gsem[b])

    start_gather(0, 0)

    @pl.loop(0, NCH, step=2)
    def _chunks(g):
        for b in range(2):
            ch = g + b
            nb = (b + 1) % 2
            nxt = ch + 1

            @pl.when(nxt < NCH)
            def _():
                @pl.when(nxt >= 2)
                def _():
                    pltpu.make_async_copy(
                        rows[nb], out_hbm.at[pl.ds(0, C)], ssem[nb]
                    ).wait()

                start_gather(nxt, nb)

            pltpu.make_async_copy(table_hbm.at[idx[b]], rows[b], gsem[b]).wait()

            @pl.loop(0, C, unroll=4)
            def _scale(r):
                for k in range(D // 16):
                    sl = pl.ds(k * 16, 16)
                    rows[b][r, sl] = rows[b][r, sl] * SCALE

            pltpu.async_copy(
                rows[b], out_hbm.at[pl.ds(base + ch * C, C)], ssem[b]
            )

    for b in range(2):
        pltpu.make_async_copy(rows[b], out_hbm.at[pl.ds(0, C)], ssem[b]).wait()


_lookup = pl.kernel(
    _body,
    out_type=jax.ShapeDtypeStruct((B, D), jnp.float32),
    mesh=_mesh,
    scratch_types=[
        pltpu.VMEM((C,), jnp.int32),
        pltpu.VMEM((C,), jnp.int32),
        pltpu.VMEM((C, D), jnp.float32),
        pltpu.VMEM((C, D), jnp.float32),
        pltpu.SemaphoreType.DMA,
        pltpu.SemaphoreType.DMA,
        pltpu.SemaphoreType.DMA,
        pltpu.SemaphoreType.DMA,
    ],
    compiler_params=pltpu.CompilerParams(use_tc_tiling_on_sc=False),
)


@jax.jit
def kernel(x, table):
    flat = x.reshape(B)
    out = _lookup(table, flat)
    return out.reshape(ROWS, COLS, D)
